# baseline (speedup 1.0000x reference)
import functools

import jax
import jax.numpy as jnp
import numpy as np
from jax import lax
from jax.experimental import pallas as pl
from jax.experimental.pallas import tpu as pltpu
from jax.experimental.pallas import tpu_sc as plsc

V = 1000000
D = 32
NC, NS = 2, 16
NW = NC * NS
NB_FULL = 7812
TAIL = V - NB_FULL * 128
PER_W = 244
UNIT = 128
CHUNK_UNITS = 10


def _detile_body(tab_t, tail1d, consts, scratch, stage, obuf, cv, sem):
    wid = lax.axis_index("s") * NC + lax.axis_index("c")
    start = wid * PER_W
    pltpu.sync_copy(consts, cv)
    p32 = [cv[r] for r in range(8)]

    def do_block(vb, width):
        for fb in range(4):
            pltpu.sync_copy(
                tab_t.at[pl.ds(fb * 8, 8), pl.ds(vb * 128, width)],
                stage.at[fb, :, pl.ds(0, width)])
        for c in range(D):
            for vh in range(width // 16):
                val = stage[c // 8, c % 8, pl.ds(vh * 16, 16)]
                base = vh * 512 + (c // 8) * 8
                plsc.store_scatter(obuf.at[pl.ds(base, 488)], [p32[c % 8]], val)
        pltpu.async_copy(
            obuf.at[pl.ds(0, width * D)],
            scratch.at[pl.ds(vb * 128 * D, width * D)], sem).wait()

    def step(i, carry):
        do_block(start + i, 128)
        return carry

    lax.fori_loop(0, PER_W, step, 0)

    @pl.when(wid < 4)
    def _():
        do_block(NW * PER_W + wid, 128)

    @pl.when(wid == NW - 1)
    def _():
        pltpu.sync_copy(tail1d, obuf.at[pl.ds(0, TAIL * D)])
        pltpu.async_copy(
            obuf.at[pl.ds(0, TAIL * D)],
            scratch.at[pl.ds(NB_FULL * 128 * D, TAIL * D)], sem).wait()


def _gather_body(idx_hbm, scr2d, consts, out4, idx_v, rows_v, tv, cv,
                 gsem0, gsem1, osem):
    wid = lax.axis_index("s") * NC + lax.axis_index("c")
    units_per_w = (50 * 16384 // UNIT) // NW
    u_base = wid * units_per_w
    n_chunks = units_per_w // CHUNK_UNITS
    gsem = (gsem0, gsem1)
    pltpu.sync_copy(consts, cv)
    p128 = [cv[8 + r] for r in range(8)]

    def start_chunk(g, p):
        off = (u_base + g * CHUNK_UNITS) * UNIT
        pltpu.sync_copy(idx_hbm.at[pl.ds(off, CHUNK_UNITS * UNIT)], idx_v.at[p])
        pltpu.async_copy(scr2d.at[idx_v.at[p]], rows_v.at[p], gsem[p])

    def wait_chunk(p):
        pltpu.make_async_copy(
            scr2d.at[idx_v.at[p]], rows_v.at[p], gsem[p]).wait()

    def do_chunk(g, p):
        def unit_step(j, carry):
            u = u_base + g * CHUNK_UNITS + j
            t = u // 128
            bb = u % 128
            for bi in range(UNIT):
                for c0 in (0, 16):
                    val = rows_v[p, j * UNIT + bi, pl.ds(c0, 16)]
                    base = c0 * 128 + (bi // 8) * 8
                    plsc.store_scatter(
                        tv.at[pl.ds(base, 1928)], [p128[bi % 8]], val)
            for cb in range(4):
                pltpu.async_copy(
                    tv.at[pl.ds(cb * 1024, 1024)], out4.at[t, cb, bb],
                    osem).wait()
            return carry

        lax.fori_loop(0, CHUNK_UNITS, unit_step, 0)

    n_pairs = n_chunks // 2
    start_chunk(0, 0)

    def pair_step(h, carry):
        g0 = 2 * h
        start_chunk(g0 + 1, 1)
        wait_chunk(0)
        do_chunk(g0, 0)

        @pl.when(h + 1 < n_pairs)
        def _():
            start_chunk(g0 + 2, 0)

        wait_chunk(1)
        do_chunk(g0 + 1, 1)
        return carry

    lax.fori_loop(0, n_pairs, pair_step, 0)


@jax.jit
def _emb(x, table):
    mesh = plsc.VectorSubcoreMesh(core_axis_name="c", subcore_axis_name="s")

    detile = functools.partial(
        pl.kernel,
        mesh=mesh,
        out_type=jax.ShapeDtypeStruct((V * D,), jnp.float32),
        scratch_types=[
            pltpu.VMEM((4, 8, 128), jnp.float32),
            pltpu.VMEM((128 * D,), jnp.float32),
            pltpu.VMEM((16, 16), jnp.int32),
            pltpu.SemaphoreType.DMA,
        ],
        compiler_params=pltpu.CompilerParams(needs_layout_passes=False),
    )(_detile_body)

    gather = functools.partial(
        pl.kernel,
        mesh=mesh,
        out_type=jax.ShapeDtypeStruct((50, 4, 128, 1024), jnp.float32),
        scratch_types=[
            pltpu.VMEM((2, CHUNK_UNITS * UNIT), jnp.int32),
            pltpu.VMEM((2, CHUNK_UNITS * UNIT, D), jnp.float32),
            pltpu.VMEM((4 * 1024,), jnp.float32),
            pltpu.VMEM((16, 16), jnp.int32),
            pltpu.SemaphoreType.DMA,
            pltpu.SemaphoreType.DMA,
            pltpu.SemaphoreType.DMA,
        ],
        compiler_params=pltpu.CompilerParams(
            use_tc_tiling_on_sc=False, needs_layout_passes=False),
    )(_gather_body)

    lane = np.arange(16, dtype=np.int32)
    consts = jnp.asarray(
        np.stack([lane * 32 + r for r in range(8)]
                 + [lane * 128 + r for r in range(8)]))
    tab_t = jnp.transpose(table)
    tail1d = table[NB_FULL * 128:, :].reshape(TAIL * D)
    scratch = detile(tab_t, tail1d, consts)
    scr2d = scratch.reshape(V, D)
    idx_t = jnp.transpose(x).reshape(50 * 16384)
    out4 = gather(idx_t, scr2d, consts)
    out5 = out4.reshape(50, 4, 128, 8, 128)
    return jnp.transpose(out5, (2, 4, 0, 1, 3)).reshape(16384, 50, D)


def kernel(x, table):
    return _emb(x, table)


# speedup vs baseline: 1.3598x; 1.3598x over previous
import functools

import jax
import jax.numpy as jnp
import numpy as np
from jax import lax
from jax.experimental import pallas as pl
from jax.experimental.pallas import tpu as pltpu
from jax.experimental.pallas import tpu_sc as plsc

V = 1000000
D = 32
NC, NS = 2, 16
NW = NC * NS
NB_FULL = 7812
TAIL = V - NB_FULL * 128
PER_W = 244
UNIT = 128
CHUNK_UNITS = 10


def _detile_body(tab_t, tail1d, consts, scratch, stage0, stage1, obuf0,
                 obuf1, cv, sem, osem0, osem1):
    wid = lax.axis_index("s") * NC + lax.axis_index("c")
    start = wid * PER_W
    osem = (osem0, osem1)
    stage = (stage0, stage1)
    obuf = (obuf0, obuf1)
    pltpu.sync_copy(consts, cv)
    p32 = [cv[r] for r in range(8)]

    def do_block(vb, width, q):
        for fb in range(4):
            pltpu.async_copy(
                tab_t.at[pl.ds(fb * 8, 8), pl.ds(vb * 128, width)],
                stage[q].at[fb, :, pl.ds(0, width)], sem)
        for fb in range(4):
            pltpu.make_async_copy(
                tab_t.at[pl.ds(fb * 8, 8), pl.ds(vb * 128, width)],
                stage[q].at[fb, :, pl.ds(0, width)], sem).wait()
        for c in range(D):
            for vh in range(width // 16):
                val = stage[q][c // 8, c % 8, pl.ds(vh * 16, 16)]
                base = vh * 512 + (c // 8) * 8
                plsc.store_scatter(
                    obuf[q].at[pl.ds(base, 488)], [p32[c % 8]], val)
        pltpu.async_copy(
            obuf[q].at[pl.ds(0, width * D)],
            scratch.at[pl.ds(vb * 128 * D, width * D)], osem[q])

    def wait_out(q, width):
        pltpu.make_async_copy(
            obuf[q].at[pl.ds(0, width * D)],
            scratch.at[pl.ds(0, width * D)], osem[q]).wait()

    def step(i, carry):
        @pl.when(i >= 1)
        def _():
            wait_out(0, 128)
        do_block(start + 2 * i, 128, 0)

        @pl.when(i >= 1)
        def _():
            wait_out(1, 128)
        do_block(start + 2 * i + 1, 128, 1)
        return carry

    lax.fori_loop(0, PER_W // 2, step, 0)
    wait_out(0, 128)
    wait_out(1, 128)

    @pl.when(wid < 4)
    def _():
        do_block(NW * PER_W + wid, 128, 0)
        wait_out(0, 128)

    @pl.when(wid == NW - 1)
    def _():
        pltpu.sync_copy(tail1d, obuf0.at[pl.ds(0, TAIL * D)])
        pltpu.async_copy(
            obuf0.at[pl.ds(0, TAIL * D)],
            scratch.at[pl.ds(NB_FULL * 128 * D, TAIL * D)], sem).wait()


def _gather_body(idx_hbm, scr2d, consts, out4, idx_v0, idx_v1, rows_v0,
                 rows_v1, tv0, tv1, cv, gsem0, gsem1, osem0, osem1):
    wid = lax.axis_index("s") * NC + lax.axis_index("c")
    units_per_w = (50 * 16384 // UNIT) // NW
    u_base = wid * units_per_w
    n_chunks = units_per_w // CHUNK_UNITS
    gsem = (gsem0, gsem1)
    osem = (osem0, osem1)
    idx_v = (idx_v0, idx_v1)
    rows_v = (rows_v0, rows_v1)
    tv = (tv0, tv1)
    pltpu.sync_copy(consts, cv)
    p128 = [cv[8 + r] for r in range(8)]

    def start_chunk(g, p):
        off = (u_base + g * CHUNK_UNITS) * UNIT
        pltpu.sync_copy(idx_hbm.at[pl.ds(off, CHUNK_UNITS * UNIT)], idx_v[p])
        pltpu.async_copy(scr2d.at[idx_v[p]], rows_v[p], gsem[p])

    def wait_chunk(p):
        pltpu.make_async_copy(
            scr2d.at[idx_v[p]], rows_v[p], gsem[p]).wait()

    def do_unit(g, p, j, q):
        u = u_base + g * CHUNK_UNITS + j
        t = u // 128
        bb = u % 128
        for bi in range(UNIT):
            for c0 in (0, 16):
                val = rows_v[p][j * UNIT + bi, pl.ds(c0, 16)]
                base = c0 * 128 + (bi // 8) * 8
                plsc.store_scatter(
                    tv[q].at[pl.ds(base, 1928)], [p128[bi % 8]], val)
        for cb in range(4):
            pltpu.async_copy(
                tv[q].at[pl.ds(cb * 1024, 1024)], out4.at[t, cb, bb], osem[q])

    def wait_unit(q):
        for cb in range(4):
            pltpu.make_async_copy(
                tv[q].at[pl.ds(cb * 1024, 1024)], out4.at[0, cb, 0],
                osem[q]).wait()

    def do_chunk(g, p):
        def unit_step(h, carry):
            @pl.when(h >= 1)
            def _():
                wait_unit(0)
            do_unit(g, p, 2 * h, 0)

            @pl.when(h >= 1)
            def _():
                wait_unit(1)
            do_unit(g, p, 2 * h + 1, 1)
            return carry

        lax.fori_loop(0, CHUNK_UNITS // 2, unit_step, 0)
        wait_unit(0)
        wait_unit(1)

    n_pairs = n_chunks // 2
    start_chunk(0, 0)

    def pair_step(h, carry):
        g0 = 2 * h
        start_chunk(g0 + 1, 1)
        wait_chunk(0)
        do_chunk(g0, 0)

        @pl.when(h + 1 < n_pairs)
        def _():
            start_chunk(g0 + 2, 0)

        wait_chunk(1)
        do_chunk(g0 + 1, 1)
        return carry

    lax.fori_loop(0, n_pairs, pair_step, 0)


@jax.jit
def _emb(x, table):
    mesh = plsc.VectorSubcoreMesh(core_axis_name="c", subcore_axis_name="s")

    detile = functools.partial(
        pl.kernel,
        mesh=mesh,
        out_type=jax.ShapeDtypeStruct((V * D,), jnp.float32),
        scratch_types=[
            pltpu.VMEM((4, 8, 128), jnp.float32),
            pltpu.VMEM((4, 8, 128), jnp.float32),
            pltpu.VMEM((128 * D,), jnp.float32),
            pltpu.VMEM((128 * D,), jnp.float32),
            pltpu.VMEM((16, 16), jnp.int32),
            pltpu.SemaphoreType.DMA,
            pltpu.SemaphoreType.DMA,
            pltpu.SemaphoreType.DMA,
        ],
        compiler_params=pltpu.CompilerParams(needs_layout_passes=False),
    )(_detile_body)

    gather = functools.partial(
        pl.kernel,
        mesh=mesh,
        out_type=jax.ShapeDtypeStruct((50, 4, 128, 1024), jnp.float32),
        scratch_types=[
            pltpu.VMEM((CHUNK_UNITS * UNIT,), jnp.int32),
            pltpu.VMEM((CHUNK_UNITS * UNIT,), jnp.int32),
            pltpu.VMEM((CHUNK_UNITS * UNIT, D), jnp.float32),
            pltpu.VMEM((CHUNK_UNITS * UNIT, D), jnp.float32),
            pltpu.VMEM((4 * 1024,), jnp.float32),
            pltpu.VMEM((4 * 1024,), jnp.float32),
            pltpu.VMEM((16, 16), jnp.int32),
            pltpu.SemaphoreType.DMA,
            pltpu.SemaphoreType.DMA,
            pltpu.SemaphoreType.DMA,
            pltpu.SemaphoreType.DMA,
        ],
        compiler_params=pltpu.CompilerParams(
            use_tc_tiling_on_sc=False, needs_layout_passes=False),
    )(_gather_body)

    lane = np.arange(16, dtype=np.int32)
    consts = jnp.asarray(
        np.stack([lane * 32 + r for r in range(8)]
                 + [lane * 128 + r for r in range(8)]))
    tab_t = jnp.transpose(table)
    tail1d = table[NB_FULL * 128:, :].reshape(TAIL * D)
    scratch = detile(tab_t, tail1d, consts)
    scr2d = scratch.reshape(V, D)
    idx_t = jnp.transpose(x).reshape(50 * 16384)
    out4 = gather(idx_t, scr2d, consts)
    out5 = out4.reshape(50, 4, 128, 8, 128)
    return jnp.transpose(out5, (2, 4, 0, 1, 3)).reshape(16384, 50, D)


def kernel(x, table):
    return _emb(x, table)


# speedup vs baseline: 1.4518x; 1.0677x over previous
import functools

import jax
import jax.numpy as jnp
import numpy as np
from jax import lax
from jax.experimental import pallas as pl
from jax.experimental.pallas import tpu as pltpu
from jax.experimental.pallas import tpu_sc as plsc

V = 1000000
D = 32
NC, NS = 2, 16
NW = NC * NS
NB_FULL = 7812
TAIL = V - NB_FULL * 128
PER_W = 244
UNIT = 128
CHUNK_UNITS = 10


def _detile_body(tab_t, tail1d, consts, scratch, stage0, stage1, obuf0,
                 obuf1, cv, sem, osem0, osem1):
    wid = lax.axis_index("s") * NC + lax.axis_index("c")
    start = wid * PER_W
    osem = (osem0, osem1)
    stage = (stage0, stage1)
    obuf = (obuf0, obuf1)
    pltpu.sync_copy(consts, cv)
    p32 = [cv[r] for r in range(8)]

    def do_block(vb, width, q):
        for fb in range(4):
            pltpu.async_copy(
                tab_t.at[pl.ds(fb * 8, 8), pl.ds(vb * 128, width)],
                stage[q].at[fb, :, pl.ds(0, width)], sem)
        for fb in range(4):
            pltpu.make_async_copy(
                tab_t.at[pl.ds(fb * 8, 8), pl.ds(vb * 128, width)],
                stage[q].at[fb, :, pl.ds(0, width)], sem).wait()
        pairs = [(c, vh) for c in range(D) for vh in range(width // 16)]
        for k in range(0, len(pairs), 8):
            batch = pairs[k:k + 8]
            vals = [stage[q][c // 8, c % 8, pl.ds(vh * 16, 16)]
                    for c, vh in batch]
            for (c, vh), val in zip(batch, vals):
                base = vh * 512 + (c // 8) * 8
                plsc.store_scatter(
                    obuf[q].at[pl.ds(base, 488)], [p32[c % 8]], val)
        pltpu.async_copy(
            obuf[q].at[pl.ds(0, width * D)],
            scratch.at[pl.ds(vb * 128 * D, width * D)], osem[q])

    def wait_out(q, width):
        pltpu.make_async_copy(
            obuf[q].at[pl.ds(0, width * D)],
            scratch.at[pl.ds(0, width * D)], osem[q]).wait()

    def step(i, carry):
        @pl.when(i >= 1)
        def _():
            wait_out(0, 128)
        do_block(start + 2 * i, 128, 0)

        @pl.when(i >= 1)
        def _():
            wait_out(1, 128)
        do_block(start + 2 * i + 1, 128, 1)
        return carry

    lax.fori_loop(0, PER_W // 2, step, 0)
    wait_out(0, 128)
    wait_out(1, 128)

    @pl.when(wid < 4)
    def _():
        do_block(NW * PER_W + wid, 128, 0)
        wait_out(0, 128)

    @pl.when(wid == NW - 1)
    def _():
        pltpu.sync_copy(tail1d, obuf0.at[pl.ds(0, TAIL * D)])
        pltpu.async_copy(
            obuf0.at[pl.ds(0, TAIL * D)],
            scratch.at[pl.ds(NB_FULL * 128 * D, TAIL * D)], sem).wait()


def _gather_body(idx_hbm, scr2d, consts, out4, idx_v0, idx_v1, rows_v0,
                 rows_v1, tv0, tv1, cv, gsem0, gsem1, osem0, osem1):
    wid = lax.axis_index("s") * NC + lax.axis_index("c")
    units_per_w = (50 * 16384 // UNIT) // NW
    u_base = wid * units_per_w
    n_chunks = units_per_w // CHUNK_UNITS
    gsem = (gsem0, gsem1)
    osem = (osem0, osem1)
    idx_v = (idx_v0, idx_v1)
    rows_v = (rows_v0, rows_v1)
    tv = (tv0, tv1)
    pltpu.sync_copy(consts, cv)
    p128 = [cv[8 + r] for r in range(8)]

    def start_chunk(g, p):
        off = (u_base + g * CHUNK_UNITS) * UNIT
        pltpu.sync_copy(idx_hbm.at[pl.ds(off, CHUNK_UNITS * UNIT)], idx_v[p])
        pltpu.async_copy(scr2d.at[idx_v[p]], rows_v[p], gsem[p])

    def wait_chunk(p):
        pltpu.make_async_copy(
            scr2d.at[idx_v[p]], rows_v[p], gsem[p]).wait()

    def do_unit(g, p, j, q):
        u = u_base + g * CHUNK_UNITS + j
        t = u // 128
        bb = u % 128
        pairs = [(bi, c0) for bi in range(UNIT) for c0 in (0, 16)]
        for k in range(0, len(pairs), 8):
            batch = pairs[k:k + 8]
            vals = [rows_v[p][j * UNIT + bi, pl.ds(c0, 16)]
                    for bi, c0 in batch]
            for (bi, c0), val in zip(batch, vals):
                base = c0 * 128 + (bi // 8) * 8
                plsc.store_scatter(
                    tv[q].at[pl.ds(base, 1928)], [p128[bi % 8]], val)
        for cb in range(4):
            pltpu.async_copy(
                tv[q].at[pl.ds(cb * 1024, 1024)], out4.at[t, cb, bb], osem[q])

    def wait_unit(q):
        for cb in range(4):
            pltpu.make_async_copy(
                tv[q].at[pl.ds(cb * 1024, 1024)], out4.at[0, cb, 0],
                osem[q]).wait()

    def do_chunk(g, p):
        def unit_step(h, carry):
            @pl.when(h >= 1)
            def _():
                wait_unit(0)
            do_unit(g, p, 2 * h, 0)

            @pl.when(h >= 1)
            def _():
                wait_unit(1)
            do_unit(g, p, 2 * h + 1, 1)
            return carry

        lax.fori_loop(0, CHUNK_UNITS // 2, unit_step, 0)
        wait_unit(0)
        wait_unit(1)

    n_pairs = n_chunks // 2
    start_chunk(0, 0)

    def pair_step(h, carry):
        g0 = 2 * h
        start_chunk(g0 + 1, 1)
        wait_chunk(0)
        do_chunk(g0, 0)

        @pl.when(h + 1 < n_pairs)
        def _():
            start_chunk(g0 + 2, 0)

        wait_chunk(1)
        do_chunk(g0 + 1, 1)
        return carry

    lax.fori_loop(0, n_pairs, pair_step, 0)


@jax.jit
def _emb(x, table):
    mesh = plsc.VectorSubcoreMesh(core_axis_name="c", subcore_axis_name="s")

    detile = functools.partial(
        pl.kernel,
        mesh=mesh,
        out_type=jax.ShapeDtypeStruct((V * D,), jnp.float32),
        scratch_types=[
            pltpu.VMEM((4, 8, 128), jnp.float32),
            pltpu.VMEM((4, 8, 128), jnp.float32),
            pltpu.VMEM((128 * D,), jnp.float32),
            pltpu.VMEM((128 * D,), jnp.float32),
            pltpu.VMEM((16, 16), jnp.int32),
            pltpu.SemaphoreType.DMA,
            pltpu.SemaphoreType.DMA,
            pltpu.SemaphoreType.DMA,
        ],
        compiler_params=pltpu.CompilerParams(needs_layout_passes=False),
    )(_detile_body)

    gather = functools.partial(
        pl.kernel,
        mesh=mesh,
        out_type=jax.ShapeDtypeStruct((50, 4, 128, 1024), jnp.float32),
        scratch_types=[
            pltpu.VMEM((CHUNK_UNITS * UNIT,), jnp.int32),
            pltpu.VMEM((CHUNK_UNITS * UNIT,), jnp.int32),
            pltpu.VMEM((CHUNK_UNITS * UNIT, D), jnp.float32),
            pltpu.VMEM((CHUNK_UNITS * UNIT, D), jnp.float32),
            pltpu.VMEM((4 * 1024,), jnp.float32),
            pltpu.VMEM((4 * 1024,), jnp.float32),
            pltpu.VMEM((16, 16), jnp.int32),
            pltpu.SemaphoreType.DMA,
            pltpu.SemaphoreType.DMA,
            pltpu.SemaphoreType.DMA,
            pltpu.SemaphoreType.DMA,
        ],
        compiler_params=pltpu.CompilerParams(
            use_tc_tiling_on_sc=False, needs_layout_passes=False),
    )(_gather_body)

    lane = np.arange(16, dtype=np.int32)
    consts = jnp.asarray(
        np.stack([lane * 32 + r for r in range(8)]
                 + [lane * 128 + r for r in range(8)]))
    tab_t = jnp.transpose(table)
    tail1d = table[NB_FULL * 128:, :].reshape(TAIL * D)
    scratch = detile(tab_t, tail1d, consts)
    scr2d = scratch.reshape(V, D)
    idx_t = jnp.transpose(x).reshape(50 * 16384)
    out4 = gather(idx_t, scr2d, consts)
    out5 = out4.reshape(50, 4, 128, 8, 128)
    return jnp.transpose(out5, (2, 4, 0, 1, 3)).reshape(16384, 50, D)


def kernel(x, table):
    return _emb(x, table)


# speedup vs baseline: 1.6879x; 1.1627x over previous
import functools

import jax
import jax.numpy as jnp
import numpy as np
from jax import lax
from jax.experimental import pallas as pl
from jax.experimental.pallas import tpu as pltpu
from jax.experimental.pallas import tpu_sc as plsc

V = 1000000
D = 32
NC, NS = 2, 16
NW = NC * NS
NB_FULL = 7812
TAIL = V - NB_FULL * 128
PER_W = 244
UNIT = 128
CHUNK_UNITS = 10


def _detile_body(tab_t, tail1d, consts, scratch, stage0, stage1, obuf0,
                 obuf1, cv, sem, osem0, osem1, ssem0, ssem1):
    wid = lax.axis_index("s") * NC + lax.axis_index("c")
    start = wid * PER_W
    osem = (osem0, osem1)
    ssem = (ssem0, ssem1)
    stage = (stage0, stage1)
    obuf = (obuf0, obuf1)
    pltpu.sync_copy(consts, cv)
    p32 = [cv[r] for r in range(8)]

    def issue_stage(vb, width, q):
        for fb in range(4):
            pltpu.async_copy(
                tab_t.at[pl.ds(fb * 8, 8), pl.ds(vb * 128, width)],
                stage[q].at[fb, :, pl.ds(0, width)], ssem[q])

    def wait_stage(width, q):
        for fb in range(4):
            pltpu.make_async_copy(
                tab_t.at[pl.ds(fb * 8, 8), pl.ds(0, width)],
                stage[q].at[fb, :, pl.ds(0, width)], ssem[q]).wait()

    def do_block(vb, width, q):
        wait_stage(width, q)
        pairs = [(c, vh) for c in range(D) for vh in range(width // 16)]
        for k in range(0, len(pairs), 8):
            batch = pairs[k:k + 8]
            vals = [stage[q][c // 8, c % 8, pl.ds(vh * 16, 16)]
                    for c, vh in batch]
            for (c, vh), val in zip(batch, vals):
                base = vh * 512 + (c // 8) * 8
                plsc.store_scatter(
                    obuf[q].at[pl.ds(base, 488)], [p32[c % 8]], val)
        pltpu.async_copy(
            obuf[q].at[pl.ds(0, width * D)],
            scratch.at[pl.ds(vb * 128 * D, width * D)], osem[q])

    def wait_out(q, width):
        pltpu.make_async_copy(
            obuf[q].at[pl.ds(0, width * D)],
            scratch.at[pl.ds(0, width * D)], osem[q]).wait()

    n2 = PER_W // 2
    issue_stage(start, 128, 0)
    issue_stage(start + 1, 128, 1)

    def step(i, carry):
        @pl.when(i >= 1)
        def _():
            wait_out(0, 128)
        do_block(start + 2 * i, 128, 0)

        @pl.when(i + 1 < n2)
        def _():
            issue_stage(start + 2 * i + 2, 128, 0)

        @pl.when(i >= 1)
        def _():
            wait_out(1, 128)
        do_block(start + 2 * i + 1, 128, 1)

        @pl.when(i + 1 < n2)
        def _():
            issue_stage(start + 2 * i + 3, 128, 1)
        return carry

    lax.fori_loop(0, n2, step, 0)
    wait_out(0, 128)
    wait_out(1, 128)

    @pl.when(wid < 4)
    def _():
        issue_stage(NW * PER_W + wid, 128, 0)
        do_block(NW * PER_W + wid, 128, 0)
        wait_out(0, 128)

    @pl.when(wid == NW - 1)
    def _():
        pltpu.sync_copy(tail1d, obuf0.at[pl.ds(0, TAIL * D)])
        pltpu.async_copy(
            obuf0.at[pl.ds(0, TAIL * D)],
            scratch.at[pl.ds(NB_FULL * 128 * D, TAIL * D)], sem).wait()


def _gather_body(idx_hbm, scr2d, consts, out4, idx_v0, idx_v1, rows_v0,
                 rows_v1, tv0, tv1, cv, gsem0, gsem1, osem0, osem1):
    wid = lax.axis_index("s") * NC + lax.axis_index("c")
    units_per_w = (50 * 16384 // UNIT) // NW
    u_base = wid * units_per_w
    n_chunks = units_per_w // CHUNK_UNITS
    gsem = (gsem0, gsem1)
    osem = (osem0, osem1)
    idx_v = (idx_v0, idx_v1)
    rows_v = (rows_v0, rows_v1)
    tv = (tv0, tv1)
    pltpu.sync_copy(consts, cv)
    p128 = [cv[8 + r] for r in range(8)]

    def start_chunk(g, p):
        off = (u_base + g * CHUNK_UNITS) * UNIT
        pltpu.sync_copy(idx_hbm.at[pl.ds(off, CHUNK_UNITS * UNIT)], idx_v[p])
        pltpu.async_copy(scr2d.at[idx_v[p]], rows_v[p], gsem[p])

    def wait_chunk(p):
        pltpu.make_async_copy(
            scr2d.at[idx_v[p]], rows_v[p], gsem[p]).wait()

    def do_unit(g, p, j, q):
        u = u_base + g * CHUNK_UNITS + j
        t = u // 128
        bb = u % 128
        pairs = [(bi, c0) for bi in range(UNIT) for c0 in (0, 16)]
        for k in range(0, len(pairs), 8):
            batch = pairs[k:k + 8]
            vals = [rows_v[p][j * UNIT + bi, pl.ds(c0, 16)]
                    for bi, c0 in batch]
            for (bi, c0), val in zip(batch, vals):
                base = c0 * 128 + (bi // 8) * 8
                plsc.store_scatter(
                    tv[q].at[pl.ds(base, 1928)], [p128[bi % 8]], val)
        for cb in range(4):
            pltpu.async_copy(
                tv[q].at[pl.ds(cb * 1024, 1024)], out4.at[t, cb, bb], osem[q])

    def wait_unit(q):
        for cb in range(4):
            pltpu.make_async_copy(
                tv[q].at[pl.ds(cb * 1024, 1024)], out4.at[0, cb, 0],
                osem[q]).wait()

    def do_chunk(g, p):
        def unit_step(h, carry):
            @pl.when(h >= 1)
            def _():
                wait_unit(0)
            do_unit(g, p, 2 * h, 0)

            @pl.when(h >= 1)
            def _():
                wait_unit(1)
            do_unit(g, p, 2 * h + 1, 1)
            return carry

        lax.fori_loop(0, CHUNK_UNITS // 2, unit_step, 0)
        wait_unit(0)
        wait_unit(1)

    n_pairs = n_chunks // 2
    start_chunk(0, 0)

    def pair_step(h, carry):
        g0 = 2 * h
        start_chunk(g0 + 1, 1)
        wait_chunk(0)
        do_chunk(g0, 0)

        @pl.when(h + 1 < n_pairs)
        def _():
            start_chunk(g0 + 2, 0)

        wait_chunk(1)
        do_chunk(g0 + 1, 1)
        return carry

    lax.fori_loop(0, n_pairs, pair_step, 0)


@jax.jit
def _emb(x, table):
    mesh = plsc.VectorSubcoreMesh(core_axis_name="c", subcore_axis_name="s")

    detile = functools.partial(
        pl.kernel,
        mesh=mesh,
        out_type=jax.ShapeDtypeStruct((V * D,), jnp.float32),
        scratch_types=[
            pltpu.VMEM((4, 8, 128), jnp.float32),
            pltpu.VMEM((4, 8, 128), jnp.float32),
            pltpu.VMEM((128 * D,), jnp.float32),
            pltpu.VMEM((128 * D,), jnp.float32),
            pltpu.VMEM((16, 16), jnp.int32),
            pltpu.SemaphoreType.DMA,
            pltpu.SemaphoreType.DMA,
            pltpu.SemaphoreType.DMA,
            pltpu.SemaphoreType.DMA,
            pltpu.SemaphoreType.DMA,
        ],
        compiler_params=pltpu.CompilerParams(needs_layout_passes=False),
    )(_detile_body)

    gather = functools.partial(
        pl.kernel,
        mesh=mesh,
        out_type=jax.ShapeDtypeStruct((50, 4, 128, 1024), jnp.float32),
        scratch_types=[
            pltpu.VMEM((CHUNK_UNITS * UNIT,), jnp.int32),
            pltpu.VMEM((CHUNK_UNITS * UNIT,), jnp.int32),
            pltpu.VMEM((CHUNK_UNITS * UNIT, D), jnp.float32),
            pltpu.VMEM((CHUNK_UNITS * UNIT, D), jnp.float32),
            pltpu.VMEM((4 * 1024,), jnp.float32),
            pltpu.VMEM((4 * 1024,), jnp.float32),
            pltpu.VMEM((16, 16), jnp.int32),
            pltpu.SemaphoreType.DMA,
            pltpu.SemaphoreType.DMA,
            pltpu.SemaphoreType.DMA,
            pltpu.SemaphoreType.DMA,
        ],
        compiler_params=pltpu.CompilerParams(
            use_tc_tiling_on_sc=False, needs_layout_passes=False),
    )(_gather_body)

    lane = np.arange(16, dtype=np.int32)
    consts = jnp.asarray(
        np.stack([lane * 32 + r for r in range(8)]
                 + [lane * 128 + r for r in range(8)]))
    tab_t = jnp.transpose(table)
    tail1d = table[NB_FULL * 128:, :].reshape(TAIL * D)
    scratch = detile(tab_t, tail1d, consts)
    scr2d = scratch.reshape(V, D)
    idx_t = jnp.transpose(x).reshape(50 * 16384)
    out4 = gather(idx_t, scr2d, consts)
    out5 = out4.reshape(50, 4, 128, 8, 128)
    return jnp.transpose(out5, (2, 4, 0, 1, 3)).reshape(16384, 50, D)


def kernel(x, table):
    return _emb(x, table)
